# baseline (speedup 1.0000x reference)
import jax
import jax.numpy as jnp
from jax import lax
from jax.experimental import pallas as pl
from jax.experimental.pallas import tpu as pltpu
from jax.experimental.pallas import tpu_sc as plsc

N = 10000
E = 320000
DIM = 128
DH = 2 * DIM

NC = 2
NS = 16
LANES = 16
CH = 80
AW = 132
EPW1 = E // (NC * NS)
EPW2 = E // NS
NCH1 = EPW1 // CH
NCH2 = EPW2 // CH
RPS = 624
RTL = N - NS * RPS

_mesh = plsc.VectorSubcoreMesh(core_axis_name="c", subcore_axis_name="s")
_SC_PARAMS = pltpu.CompilerParams(
    needs_layout_passes=False, use_tc_tiling_on_sc=False)


def _vset(dst_ref, src_ref, c):
    for j in range(CH // LANES):
        sl = pl.ds(j * LANES, LANES)
        dst_ref[sl] = src_ref[sl] + c


def _mm_body(h_ref, w_ref, b_ref, pw_ref, f_ref, gbf_ref, fbf_ref):
    ft = jnp.dot(h_ref[...], w_ref[...], preferred_element_type=jnp.float32)
    ft = ft + b_ref[...]
    g = ft * pw_ref[...]
    f_ref[0] = ft[:, :DIM]
    f_ref[1] = ft[:, DIM:]
    gbf_ref[...] = g.astype(jnp.bfloat16)
    fbf_ref[...] = ft.astype(jnp.bfloat16)


_MMB = 2000

_mm_call = pl.pallas_call(
    _mm_body,
    grid=(N // _MMB,),
    in_specs=[
        pl.BlockSpec((_MMB, DIM), lambda i: (i, 0)),
        pl.BlockSpec((DIM, DH), lambda i: (0, 0)),
        pl.BlockSpec((1, DH), lambda i: (0, 0)),
        pl.BlockSpec((1, DH), lambda i: (0, 0)),
    ],
    out_specs=[
        pl.BlockSpec((2, _MMB, DIM), lambda i: (0, i, 0)),
        pl.BlockSpec((_MMB, DH), lambda i: (i, 0)),
        pl.BlockSpec((_MMB, DH), lambda i: (i, 0)),
    ],
    out_shape=[
        jax.ShapeDtypeStruct((2, N, DIM), jnp.float32),
        jax.ShapeDtypeStruct((N, DH), jnp.bfloat16),
        jax.ShapeDtypeStruct((N, DH), jnp.bfloat16),
    ],
)


def _p1_body(gbf_hbm, fbf_hbm, src_hbm, dst_hbm, p_hbm, denp_hbm, *s):
    bufs = []
    for b in range(2):
        o = b * 5
        bufs.append(dict(
            isrc=s[o], idst=s[o + 1], ga=s[o + 2], fb=s[o + 3],
            ebuf=s[o + 4], si=s[11 + b], sr=s[13 + b], so=s[15 + b]))
    den = s[10]

    cid = lax.axis_index("c")
    sid = lax.axis_index("s")
    wid = sid * NC + cid
    base = wid * EPW1
    lane = lax.iota(jnp.int32, LANES)
    lane0 = lane == 0
    perms = [lane ^ k for k in (8, 4, 2, 1)]
    zv = jnp.zeros((LANES,), jnp.float32)

    def zden(jj, c2):
        den[pl.ds(jj * LANES, LANES)] = zv
        return c2

    lax.fori_loop(0, N // LANES, zden, 0)

    def issue_idx(g, bb):
        off = base + g * CH
        pltpu.async_copy(src_hbm.at[pl.ds(off, CH)], bb["isrc"], bb["si"])
        pltpu.async_copy(dst_hbm.at[pl.ds(off, CH)], bb["idst"], bb["si"])

    def launch_rows(bb):
        pltpu.make_async_copy(
            src_hbm.at[pl.ds(0, CH)], bb["isrc"], bb["si"]).wait()
        pltpu.make_async_copy(
            dst_hbm.at[pl.ds(0, CH)], bb["idst"], bb["si"]).wait()
        pltpu.async_copy(gbf_hbm.at[bb["isrc"]], bb["ga"], bb["sr"])
        pltpu.async_copy(fbf_hbm.at[bb["idst"]], bb["fb"], bb["sr"])

    def step(g, b):
        bb = bufs[b]
        nb = bufs[1 - b]
        pltpu.make_async_copy(gbf_hbm.at[bb["isrc"]], bb["ga"], bb["sr"]).wait()
        pltpu.make_async_copy(fbf_hbm.at[bb["idst"]], bb["fb"], bb["sr"]).wait()

        @pl.when(g + 1 < NCH1)
        def _launch_next():
            launch_rows(nb)

        @pl.when(g >= 2)
        def _drain_out():
            pltpu.make_async_copy(
                bb["ebuf"], p_hbm.at[pl.ds(0, CH)], bb["so"]).wait()

        ga, fb = bb["ga"], bb["fb"]

        def edot(jj, c2):
            sl16 = pl.ds(jj * LANES, LANES)
            dvec = bb["idst"][sl16]
            pacc = zv
            for l in range(LANES):
                i = jj * LANES + l
                acc = None
                for k in range(DH // (2 * LANES)):
                    sl = pl.ds(k * 2 * LANES, 2 * LANES)
                    a0, a1 = plsc.unpack(
                        ga[i, sl], format=plsc.PackFormat.INTERLEAVED)
                    b0, b1 = plsc.unpack(
                        fb[i, sl], format=plsc.PackFormat.INTERLEAVED)
                    t2 = a0 * b0 + a1 * b1
                    acc = t2 if acc is None else acc + t2
                for pm in perms:
                    acc = acc + acc.at[pm].get(mode="promise_in_bounds")
                acc = jnp.where(acc > 0.0, acc, 0.2 * acc)
                pva = jnp.exp(acc)
                pacc = jnp.where(lane == l, pva, pacc)
                plsc.addupdate_scatter(
                    den, [jnp.full((LANES,), dvec[l], jnp.int32)], pva,
                    mask=lane0)
            bb["ebuf"][sl16] = pacc
            return c2

        lax.fori_loop(0, CH // LANES, edot, 0)

        @pl.when(g + 2 < NCH1)
        def _prefetch_idx():
            issue_idx(g + 2, bb)

        off = base + g * CH
        pltpu.async_copy(bb["ebuf"], p_hbm.at[pl.ds(off, CH)], bb["so"])

    issue_idx(jnp.int32(0), bufs[0])
    issue_idx(jnp.int32(1), bufs[1])
    launch_rows(bufs[0])

    def pair(it, c):
        step(2 * it, 0)
        step(2 * it + 1, 1)
        return c

    lax.fori_loop(0, NCH1 // 2, pair, 0)
    step(jnp.int32(NCH1 - 1), (NCH1 - 1) % 2)
    pltpu.sync_copy(den, denp_hbm.at[pl.ds(wid * N, N)])
    for b in range(2):
        pltpu.make_async_copy(
            bufs[b]["ebuf"], p_hbm.at[pl.ds(0, CH)], bufs[b]["so"]).wait()


_p1_call = pl.kernel(
    _p1_body,
    out_type=[
        jax.ShapeDtypeStruct((E,), jnp.float32),
        jax.ShapeDtypeStruct((NC * NS * N,), jnp.float32),
    ],
    mesh=_mesh,
    compiler_params=_SC_PARAMS,
    scratch_types=[
        t for _ in range(2) for t in (
            pltpu.VMEM((CH,), jnp.int32),
            pltpu.VMEM((CH,), jnp.int32),
            pltpu.VMEM((CH, DH), jnp.bfloat16),
            pltpu.VMEM((CH, DH), jnp.bfloat16),
            pltpu.VMEM((CH,), jnp.float32),
        )
    ] + [pltpu.VMEM((N,), jnp.float32)]
      + [pltpu.SemaphoreType.DMA] * 6,
)


def _p2_body(f2_hbm, src_hbm, dst_hbm, p_hbm, zer_hbm, out_hbm, *s):
    bufs = []
    for b in range(2):
        o = b * 6
        bufs.append(dict(
            isrc=s[o], idst=s[o + 1], sdst=s[o + 2], pbuf=s[o + 3],
            rows=s[o + 4], wrows=s[o + 5],
            si=s[13 + b], sr=s[15 + b], ss=s[17 + b]))
    acc = s[12]

    cid = lax.axis_index("c")
    sid = lax.axis_index("s")
    coff = cid * N

    pltpu.sync_copy(zer_hbm.at[pl.ds(sid * RPS, RPS)],
                    acc.at[pl.ds(sid * RPS, RPS)])

    @pl.when(sid == NS - 1)
    def _zero_tail():
        pltpu.sync_copy(zer_hbm.at[pl.ds(NS * RPS, RTL)],
                        acc.at[pl.ds(NS * RPS, RTL)])

    plsc.subcore_barrier()
    base = sid * EPW2

    def issue_idx(g, bb):
        off = base + g * CH
        pltpu.async_copy(src_hbm.at[pl.ds(off, CH)], bb["isrc"], bb["si"])
        pltpu.async_copy(dst_hbm.at[pl.ds(off, CH)], bb["idst"], bb["si"])
        pltpu.async_copy(p_hbm.at[pl.ds(off, CH)], bb["pbuf"], bb["si"])

    def launch_rows(bb):
        pltpu.make_async_copy(
            src_hbm.at[pl.ds(0, CH)], bb["isrc"], bb["si"]).wait()
        pltpu.make_async_copy(
            dst_hbm.at[pl.ds(0, CH)], bb["idst"], bb["si"]).wait()
        pltpu.make_async_copy(
            p_hbm.at[pl.ds(0, CH)], bb["pbuf"], bb["si"]).wait()
        _vset(bb["isrc"], bb["isrc"], coff)
        pltpu.async_copy(f2_hbm.at[bb["isrc"]], bb["rows"], bb["sr"])

    def step(g, b):
        bb = bufs[b]
        nb = bufs[1 - b]
        pltpu.make_async_copy(
            f2_hbm.at[bb["isrc"]], bb["rows"], bb["sr"]).wait()

        @pl.when(g >= 2)
        def _drain_scatter():
            pltpu.make_async_copy(
                bb["wrows"], acc.at[bb["sdst"]], bb["ss"]).wait()

        _vset(bb["sdst"], bb["idst"], 0)

        @pl.when(g + 1 < NCH2)
        def _launch_next():
            launch_rows(nb)

        rows, wrows = bb["rows"], bb["wrows"]

        def wbody(jj, c2):
            pvec = bb["pbuf"][pl.ds(jj * LANES, LANES)]
            for l in range(LANES):
                i = jj * LANES + l
                pv = pvec[l]
                for k in range(DIM // LANES):
                    sl = pl.ds(k * LANES, LANES)
                    wrows[i, sl] = rows[i, sl] * pv
            return c2

        lax.fori_loop(0, CH // LANES, wbody, 0)

        @pl.when(g + 2 < NCH2)
        def _prefetch_idx():
            issue_idx(g + 2, bb)

        pltpu.async_copy(bb["wrows"], acc.at[bb["sdst"]], bb["ss"], add=True)

    issue_idx(jnp.int32(0), bufs[0])
    issue_idx(jnp.int32(1), bufs[1])
    launch_rows(bufs[0])

    def pair(it, c):
        step(2 * it, 0)
        step(2 * it + 1, 1)
        return c

    lax.fori_loop(0, NCH2 // 2, pair, 0)
    for b in range(2):
        pltpu.make_async_copy(
            bufs[b]["wrows"], acc.at[bufs[b]["sdst"]], bufs[b]["ss"]).wait()

    plsc.subcore_barrier()
    pltpu.sync_copy(acc.at[pl.ds(sid * RPS, RPS)],
                    out_hbm.at[pl.ds(coff + sid * RPS, RPS)])

    @pl.when(sid == NS - 1)
    def _out_tail():
        pltpu.sync_copy(acc.at[pl.ds(NS * RPS, RTL)],
                        out_hbm.at[pl.ds(coff + NS * RPS, RTL)])


_p2_call = pl.kernel(
    _p2_body,
    out_type=jax.ShapeDtypeStruct((2 * N, DIM), jnp.float32),
    mesh=_mesh,
    compiler_params=_SC_PARAMS,
    scratch_types=[
        t for _ in range(2) for t in (
            pltpu.VMEM((CH,), jnp.int32),
            pltpu.VMEM((CH,), jnp.int32),
            pltpu.VMEM((CH,), jnp.int32),
            pltpu.VMEM((CH,), jnp.float32),
            pltpu.VMEM((CH, DIM), jnp.float32),
            pltpu.VMEM((CH, DIM), jnp.float32),
        )
    ] + [pltpu.VMEM_SHARED((N, DIM), jnp.float32)]
      + [pltpu.SemaphoreType.DMA] * 6,
)


def _comb_body(a_ref, d_ref, o_ref):
    a0 = a_ref[0]
    a1 = a_ref[1]
    den = jnp.sum(d_ref[...], axis=0)[:, None] + 1e-9
    o_ref[...] = jnp.maximum(a0, a1) / den


_comb_call = pl.pallas_call(
    _comb_body,
    grid=(1,),
    in_specs=[
        pl.BlockSpec((2, N, DIM), lambda i: (0, 0, 0)),
        pl.BlockSpec((NC * NS, N), lambda i: (0, 0)),
    ],
    out_specs=pl.BlockSpec((N, DIM), lambda i: (0, 0)),
    out_shape=jax.ShapeDtypeStruct((N, DIM), jnp.float32),
)


def kernel(h_v, edge_index, fc_W, fc_b, pi_w):
    src = edge_index[0].astype(jnp.int32)
    dst = edge_index[1].astype(jnp.int32)
    f_parts, gbf, fbf = _mm_call(
        h_v, fc_W, fc_b.reshape(1, DH), pi_w.reshape(1, DH))
    f2 = f_parts.reshape(2 * N, DIM)
    p, denp = _p1_call(gbf, fbf, src, dst)
    accf = _p2_call(f2, src, dst, p, jnp.zeros((N, DIM), jnp.float32))
    return _comb_call(accf.reshape(2, N, DIM), denp.reshape(NC * NS, N))


# speedup vs baseline: 1.3913x; 1.3913x over previous
import jax
import jax.numpy as jnp
from jax import lax
from jax.experimental import pallas as pl
from jax.experimental.pallas import tpu as pltpu
from jax.experimental.pallas import tpu_sc as plsc

N = 10000
E = 320000
DIM = 128
DH = 2 * DIM

NC = 2
NS = 16
LANES = 16
CH = 80
AW = 132
EPW1 = E // (NC * NS)
EPW2 = E // NS
NCH1 = EPW1 // CH
NCH2 = EPW2 // CH
RPS = 624
RTL = N - NS * RPS

_mesh = plsc.VectorSubcoreMesh(core_axis_name="c", subcore_axis_name="s")
_SC_PARAMS = pltpu.CompilerParams(
    needs_layout_passes=False, use_tc_tiling_on_sc=False)


def _vset(dst_ref, src_ref, c):
    for j in range(CH // LANES):
        sl = pl.ds(j * LANES, LANES)
        dst_ref[sl] = src_ref[sl] + c


def _mm_body(h_ref, w_ref, b_ref, pw_ref, f_ref, gbf_ref, fbf_ref):
    ft = jnp.dot(h_ref[...], w_ref[...], preferred_element_type=jnp.float32)
    ft = ft + b_ref[...]
    g = ft * pw_ref[...]
    f_ref[0] = ft[:, :DIM]
    f_ref[1] = ft[:, DIM:]
    gbf_ref[...] = g.astype(jnp.bfloat16)
    fbf_ref[...] = ft.astype(jnp.bfloat16)


_MMB = 2000

_mm_call = pl.pallas_call(
    _mm_body,
    grid=(N // _MMB,),
    in_specs=[
        pl.BlockSpec((_MMB, DIM), lambda i: (i, 0)),
        pl.BlockSpec((DIM, DH), lambda i: (0, 0)),
        pl.BlockSpec((1, DH), lambda i: (0, 0)),
        pl.BlockSpec((1, DH), lambda i: (0, 0)),
    ],
    out_specs=[
        pl.BlockSpec((2, _MMB, DIM), lambda i: (0, i, 0)),
        pl.BlockSpec((_MMB, DH), lambda i: (i, 0)),
        pl.BlockSpec((_MMB, DH), lambda i: (i, 0)),
    ],
    out_shape=[
        jax.ShapeDtypeStruct((2, N, DIM), jnp.float32),
        jax.ShapeDtypeStruct((N, DH), jnp.bfloat16),
        jax.ShapeDtypeStruct((N, DH), jnp.bfloat16),
    ],
)


def _p1_body(gbf_hbm, fbf_hbm, src_hbm, dst_hbm, p_hbm, *s):
    bufs = []
    for b in range(2):
        o = b * 5
        bufs.append(dict(
            isrc=s[o], idst=s[o + 1], ga=s[o + 2], fb=s[o + 3],
            ebuf=s[o + 4], si=s[10 + b], sr=s[12 + b], so=s[14 + b]))

    cid = lax.axis_index("c")
    sid = lax.axis_index("s")
    wid = sid * NC + cid
    base = wid * EPW1
    lane = lax.iota(jnp.int32, LANES)
    perms = [lane ^ k for k in (8, 4, 2, 1)]
    zv = jnp.zeros((LANES,), jnp.float32)

    def issue_idx(g, bb):
        off = base + g * CH
        pltpu.async_copy(src_hbm.at[pl.ds(off, CH)], bb["isrc"], bb["si"])
        pltpu.async_copy(dst_hbm.at[pl.ds(off, CH)], bb["idst"], bb["si"])

    def launch_rows(bb):
        pltpu.make_async_copy(
            src_hbm.at[pl.ds(0, CH)], bb["isrc"], bb["si"]).wait()
        pltpu.make_async_copy(
            dst_hbm.at[pl.ds(0, CH)], bb["idst"], bb["si"]).wait()
        pltpu.async_copy(gbf_hbm.at[bb["isrc"]], bb["ga"], bb["sr"])
        pltpu.async_copy(fbf_hbm.at[bb["idst"]], bb["fb"], bb["sr"])

    def step(g, b):
        bb = bufs[b]
        nb = bufs[1 - b]
        pltpu.make_async_copy(gbf_hbm.at[bb["isrc"]], bb["ga"], bb["sr"]).wait()
        pltpu.make_async_copy(fbf_hbm.at[bb["idst"]], bb["fb"], bb["sr"]).wait()

        @pl.when(g + 1 < NCH1)
        def _launch_next():
            launch_rows(nb)

        @pl.when(g + 2 < NCH1)
        def _prefetch_idx():
            issue_idx(g + 2, bb)

        @pl.when(g >= 2)
        def _drain_out():
            pltpu.make_async_copy(
                bb["ebuf"], p_hbm.at[pl.ds(0, CH)], bb["so"]).wait()

        ga, fb = bb["ga"], bb["fb"]

        def edot(jj, c2):
            pacc = zv
            for l in range(LANES):
                i = jj * LANES + l
                acc = None
                for k in range(DH // (2 * LANES)):
                    sl = pl.ds(k * 2 * LANES, 2 * LANES)
                    a0, a1 = plsc.unpack(
                        ga[i, sl], format=plsc.PackFormat.INTERLEAVED)
                    b0, b1 = plsc.unpack(
                        fb[i, sl], format=plsc.PackFormat.INTERLEAVED)
                    t2 = a0 * b0 + a1 * b1
                    acc = t2 if acc is None else acc + t2
                for pm in perms:
                    acc = acc + acc.at[pm].get(mode="promise_in_bounds")
                pacc = jnp.where(lane == l, acc, pacc)
            bb["ebuf"][pl.ds(jj * LANES, LANES)] = pacc
            return c2

        lax.fori_loop(0, CH // LANES, edot, 0)
        for j in range(CH // LANES):
            sl = pl.ds(j * LANES, LANES)
            v = bb["ebuf"][sl]
            v = jnp.where(v > 0.0, v, 0.2 * v)
            bb["ebuf"][sl] = jnp.exp(v)
        off = base + g * CH
        pltpu.async_copy(bb["ebuf"], p_hbm.at[pl.ds(off, CH)], bb["so"])

    issue_idx(jnp.int32(0), bufs[0])
    issue_idx(jnp.int32(1), bufs[1])
    launch_rows(bufs[0])

    def pair(it, c):
        step(2 * it, 0)
        step(2 * it + 1, 1)
        return c

    lax.fori_loop(0, NCH1 // 2, pair, 0)
    step(jnp.int32(NCH1 - 1), (NCH1 - 1) % 2)
    for b in range(2):
        pltpu.make_async_copy(
            bufs[b]["ebuf"], p_hbm.at[pl.ds(0, CH)], bufs[b]["so"]).wait()


_p1_call = pl.kernel(
    _p1_body,
    out_type=jax.ShapeDtypeStruct((E,), jnp.float32),
    mesh=_mesh,
    compiler_params=_SC_PARAMS,
    scratch_types=[
        t for _ in range(2) for t in (
            pltpu.VMEM((CH,), jnp.int32),
            pltpu.VMEM((CH,), jnp.int32),
            pltpu.VMEM((CH, DH), jnp.bfloat16),
            pltpu.VMEM((CH, DH), jnp.bfloat16),
            pltpu.VMEM((CH,), jnp.float32),
        )
    ] + [pltpu.SemaphoreType.DMA] * 6,
)


def _p2_body(f2_hbm, src_hbm, dst_hbm, p_hbm, zer_hbm, zden_hbm,
             out_hbm, den_hbm, *s):
    bufs = []
    for b in range(2):
        o = b * 7
        bufs.append(dict(
            isrc=s[o], idst=s[o + 1], sdst=s[o + 2], pbuf=s[o + 3],
            sp=s[o + 4], rows=s[o + 5], wrows=s[o + 6],
            si=s[16 + b], sr=s[18 + b], ss=s[20 + b]))
    acc = s[14]
    den = s[15]

    cid = lax.axis_index("c")
    sid = lax.axis_index("s")
    coff = cid * N

    pltpu.sync_copy(zer_hbm.at[pl.ds(sid * RPS, RPS)],
                    acc.at[pl.ds(sid * RPS, RPS)])

    @pl.when(cid == 0)
    def _zero_den():
        pltpu.sync_copy(zden_hbm.at[pl.ds(sid * RPS, RPS)],
                        den.at[pl.ds(sid * RPS, RPS)])

    @pl.when(sid == NS - 1)
    def _zero_tail():
        pltpu.sync_copy(zer_hbm.at[pl.ds(NS * RPS, RTL)],
                        acc.at[pl.ds(NS * RPS, RTL)])

        @pl.when(cid == 0)
        def _zero_den_tail():
            pltpu.sync_copy(zden_hbm.at[pl.ds(NS * RPS, RTL)],
                            den.at[pl.ds(NS * RPS, RTL)])

    plsc.subcore_barrier()
    base = sid * EPW2

    def issue_idx(g, bb):
        off = base + g * CH
        pltpu.async_copy(src_hbm.at[pl.ds(off, CH)], bb["isrc"], bb["si"])
        pltpu.async_copy(dst_hbm.at[pl.ds(off, CH)], bb["idst"], bb["si"])
        pltpu.async_copy(p_hbm.at[pl.ds(off, CH)], bb["pbuf"], bb["si"])

    def launch_rows(bb):
        pltpu.make_async_copy(
            src_hbm.at[pl.ds(0, CH)], bb["isrc"], bb["si"]).wait()
        pltpu.make_async_copy(
            dst_hbm.at[pl.ds(0, CH)], bb["idst"], bb["si"]).wait()
        pltpu.make_async_copy(
            p_hbm.at[pl.ds(0, CH)], bb["pbuf"], bb["si"]).wait()
        _vset(bb["isrc"], bb["isrc"], coff)
        pltpu.async_copy(f2_hbm.at[bb["isrc"]], bb["rows"], bb["sr"])

    def step(g, b):
        bb = bufs[b]
        nb = bufs[1 - b]
        pltpu.make_async_copy(
            f2_hbm.at[bb["isrc"]], bb["rows"], bb["sr"]).wait()

        @pl.when(g >= 2)
        def _drain_scatter():
            pltpu.make_async_copy(
                bb["wrows"], acc.at[bb["sdst"]], bb["ss"]).wait()

            @pl.when(cid == 0)
            def _drain_den():
                pltpu.make_async_copy(
                    bb["sp"], den.at[bb["sdst"]], bb["ss"]).wait()

        _vset(bb["sdst"], bb["idst"], 0)
        _vset(bb["sp"], bb["pbuf"], 0)

        @pl.when(g + 1 < NCH2)
        def _launch_next():
            launch_rows(nb)

        rows, wrows = bb["rows"], bb["wrows"]

        def wbody(jj, c2):
            pvec = bb["pbuf"][pl.ds(jj * LANES, LANES)]
            for l in range(LANES):
                i = jj * LANES + l
                pv = pvec[l]
                for k in range(DIM // LANES):
                    sl = pl.ds(k * LANES, LANES)
                    wrows[i, sl] = rows[i, sl] * pv
            return c2

        lax.fori_loop(0, CH // LANES, wbody, 0)

        @pl.when(g + 2 < NCH2)
        def _prefetch_idx():
            issue_idx(g + 2, bb)

        pltpu.async_copy(bb["wrows"], acc.at[bb["sdst"]], bb["ss"], add=True)

        @pl.when(cid == 0)
        def _scatter_den():
            pltpu.async_copy(bb["sp"], den.at[bb["sdst"]], bb["ss"],
                             add=True)

    issue_idx(jnp.int32(0), bufs[0])
    issue_idx(jnp.int32(1), bufs[1])
    launch_rows(bufs[0])

    def pair(it, c):
        step(2 * it, 0)
        step(2 * it + 1, 1)
        return c

    lax.fori_loop(0, NCH2 // 2, pair, 0)
    for b in range(2):
        pltpu.make_async_copy(
            bufs[b]["wrows"], acc.at[bufs[b]["sdst"]], bufs[b]["ss"]).wait()

        @pl.when(cid == 0)
        def _drain_den_tail():
            pltpu.make_async_copy(
                bufs[b]["sp"], den.at[bufs[b]["sdst"]], bufs[b]["ss"]).wait()

    plsc.subcore_barrier()
    pltpu.sync_copy(acc.at[pl.ds(sid * RPS, RPS)],
                    out_hbm.at[pl.ds(coff + sid * RPS, RPS)])

    @pl.when(cid == 0)
    def _den_out():
        pltpu.sync_copy(den.at[pl.ds(sid * RPS, RPS)],
                        den_hbm.at[pl.ds(sid * RPS, RPS)])

    @pl.when(sid == NS - 1)
    def _out_tail():
        pltpu.sync_copy(acc.at[pl.ds(NS * RPS, RTL)],
                        out_hbm.at[pl.ds(coff + NS * RPS, RTL)])

        @pl.when(cid == 0)
        def _den_out_tail():
            pltpu.sync_copy(den.at[pl.ds(NS * RPS, RTL)],
                            den_hbm.at[pl.ds(NS * RPS, RTL)])


_p2_call = pl.kernel(
    _p2_body,
    out_type=[
        jax.ShapeDtypeStruct((2 * N, DIM), jnp.float32),
        jax.ShapeDtypeStruct((N,), jnp.float32),
    ],
    mesh=_mesh,
    compiler_params=_SC_PARAMS,
    scratch_types=[
        t for _ in range(2) for t in (
            pltpu.VMEM((CH,), jnp.int32),
            pltpu.VMEM((CH,), jnp.int32),
            pltpu.VMEM((CH,), jnp.int32),
            pltpu.VMEM((CH,), jnp.float32),
            pltpu.VMEM((CH,), jnp.float32),
            pltpu.VMEM((CH, DIM), jnp.float32),
            pltpu.VMEM((CH, DIM), jnp.float32),
        )
    ] + [pltpu.VMEM_SHARED((N, DIM), jnp.float32)]
      + [pltpu.VMEM_SHARED((N,), jnp.float32)]
      + [pltpu.SemaphoreType.DMA] * 6,
)


def _comb_body(a_ref, d_ref, o_ref):
    a0 = a_ref[0]
    a1 = a_ref[1]
    den = d_ref[...] + 1e-9
    o_ref[...] = jnp.maximum(a0, a1) / den


_comb_call = pl.pallas_call(
    _comb_body,
    grid=(1,),
    in_specs=[
        pl.BlockSpec((2, N, DIM), lambda i: (0, 0, 0)),
        pl.BlockSpec((N, 1), lambda i: (0, 0)),
    ],
    out_specs=pl.BlockSpec((N, DIM), lambda i: (0, 0)),
    out_shape=jax.ShapeDtypeStruct((N, DIM), jnp.float32),
)


def kernel(h_v, edge_index, fc_W, fc_b, pi_w):
    src = edge_index[0].astype(jnp.int32)
    dst = edge_index[1].astype(jnp.int32)
    f_parts, gbf, fbf = _mm_call(
        h_v, fc_W, fc_b.reshape(1, DH), pi_w.reshape(1, DH))
    f2 = f_parts.reshape(2 * N, DIM)
    p = _p1_call(gbf, fbf, src, dst)
    accf, den = _p2_call(f2, src, dst, p,
                         jnp.zeros((N, DIM), jnp.float32),
                         jnp.zeros((N,), jnp.float32))
    return _comb_call(accf.reshape(2, N, DIM), den.reshape(N, 1))


# speedup vs baseline: 1.4577x; 1.0477x over previous
import jax
import jax.numpy as jnp
from jax import lax
from jax.experimental import pallas as pl
from jax.experimental.pallas import tpu as pltpu
from jax.experimental.pallas import tpu_sc as plsc

N = 10000
E = 320000
DIM = 128
DH = 2 * DIM

NC = 2
NS = 16
LANES = 16
CH = 128
NCHT = E // CH
CH2 = 80
AW = 132
EPW1 = E // (NC * NS)
EPW2 = E // NS
NCH1 = EPW1 // CH
NCH2 = EPW2 // CH2
RPS = 624
RTL = N - NS * RPS

_mesh = plsc.VectorSubcoreMesh(core_axis_name="c", subcore_axis_name="s")
_SC_PARAMS = pltpu.CompilerParams(
    needs_layout_passes=False, use_tc_tiling_on_sc=False)


def _vset(dst_ref, src_ref, c):
    for j in range(CH2 // LANES):
        sl = pl.ds(j * LANES, LANES)
        dst_ref[sl] = src_ref[sl] + c


def _mm_body(h_ref, w_ref, b_ref, pw_ref, f_ref, gbf_ref, fbf_ref):
    ft = jnp.dot(h_ref[...], w_ref[...], preferred_element_type=jnp.float32)
    ft = ft + b_ref[...]
    g = ft * pw_ref[...]
    f_ref[0] = ft[:, :DIM]
    f_ref[1] = ft[:, DIM:]
    gbf_ref[...] = g.astype(jnp.bfloat16)
    fbf_ref[...] = ft.astype(jnp.bfloat16)


_MMB = 2000

_mm_call = pl.pallas_call(
    _mm_body,
    grid=(N // _MMB,),
    in_specs=[
        pl.BlockSpec((_MMB, DIM), lambda i: (i, 0)),
        pl.BlockSpec((DIM, DH), lambda i: (0, 0)),
        pl.BlockSpec((1, DH), lambda i: (0, 0)),
        pl.BlockSpec((1, DH), lambda i: (0, 0)),
    ],
    out_specs=[
        pl.BlockSpec((2, _MMB, DIM), lambda i: (0, i, 0)),
        pl.BlockSpec((_MMB, DH), lambda i: (i, 0)),
        pl.BlockSpec((_MMB, DH), lambda i: (i, 0)),
    ],
    out_shape=[
        jax.ShapeDtypeStruct((2, N, DIM), jnp.float32),
        jax.ShapeDtypeStruct((N, DH), jnp.bfloat16),
        jax.ShapeDtypeStruct((N, DH), jnp.bfloat16),
    ],
)


def _p1_body(gbf_hbm, fbf_hbm, src_hbm, dst_hbm, p_hbm, *s):
    bufs = []
    for b in range(2):
        o = b * 5
        bufs.append(dict(
            isrc=s[o], idst=s[o + 1], ga=s[o + 2], fb=s[o + 3],
            ebuf=s[o + 4], si=s[10 + b], sr=s[12 + b], so=s[14 + b]))

    cid = lax.axis_index("c")
    sid = lax.axis_index("s")
    wid = sid * NC + cid
    nw = NC * NS
    rem = NCHT - (NCHT // nw) * nw
    nch = NCHT // nw + (wid < rem).astype(jnp.int32)
    cstart = (NCHT // nw) * wid + jnp.minimum(wid, rem)
    lane = lax.iota(jnp.int32, LANES)
    perms = [lane ^ k for k in (8, 4, 2, 1)]
    zv = jnp.zeros((LANES,), jnp.float32)

    def issue_idx(g, bb):
        off = (cstart + g) * CH
        pltpu.async_copy(src_hbm.at[pl.ds(off, CH)], bb["isrc"], bb["si"])
        pltpu.async_copy(dst_hbm.at[pl.ds(off, CH)], bb["idst"], bb["si"])

    def launch_rows(bb):
        pltpu.make_async_copy(
            src_hbm.at[pl.ds(0, CH)], bb["isrc"], bb["si"]).wait()
        pltpu.make_async_copy(
            dst_hbm.at[pl.ds(0, CH)], bb["idst"], bb["si"]).wait()
        pltpu.async_copy(gbf_hbm.at[bb["isrc"]], bb["ga"], bb["sr"])
        pltpu.async_copy(fbf_hbm.at[bb["idst"]], bb["fb"], bb["sr"])

    def step(g, b):
        bb = bufs[b]
        nb = bufs[1 - b]
        pltpu.make_async_copy(gbf_hbm.at[bb["isrc"]], bb["ga"], bb["sr"]).wait()
        pltpu.make_async_copy(fbf_hbm.at[bb["idst"]], bb["fb"], bb["sr"]).wait()

        @pl.when(g + 1 < nch)
        def _launch_next():
            launch_rows(nb)

        @pl.when(g + 2 < nch)
        def _prefetch_idx():
            issue_idx(g + 2, bb)

        @pl.when(g >= 2)
        def _drain_out():
            pltpu.make_async_copy(
                bb["ebuf"], p_hbm.at[pl.ds(0, CH)], bb["so"]).wait()

        ga, fb = bb["ga"], bb["fb"]

        def edot(jj, c2):
            pacc = zv
            for l in range(LANES):
                i = jj * LANES + l
                acc = None
                for k in range(DH // (2 * LANES)):
                    sl = pl.ds(k * 2 * LANES, 2 * LANES)
                    a0, a1 = plsc.unpack(
                        ga[i, sl], format=plsc.PackFormat.INTERLEAVED)
                    b0, b1 = plsc.unpack(
                        fb[i, sl], format=plsc.PackFormat.INTERLEAVED)
                    t2 = a0 * b0 + a1 * b1
                    acc = t2 if acc is None else acc + t2
                for pm in perms:
                    acc = acc + acc.at[pm].get(mode="promise_in_bounds")
                pacc = jnp.where(lane == l, acc, pacc)
            bb["ebuf"][pl.ds(jj * LANES, LANES)] = pacc
            return c2

        lax.fori_loop(0, CH // LANES, edot, 0)
        for j in range(CH // LANES):
            sl = pl.ds(j * LANES, LANES)
            v = bb["ebuf"][sl]
            v = jnp.where(v > 0.0, v, 0.2 * v)
            bb["ebuf"][sl] = jnp.exp(v)
        off = (cstart + g) * CH
        pltpu.async_copy(bb["ebuf"], p_hbm.at[pl.ds(off, CH)], bb["so"])

    issue_idx(jnp.int32(0), bufs[0])
    issue_idx(jnp.int32(1), bufs[1])
    launch_rows(bufs[0])

    def pair(it, c):
        step(2 * it, 0)
        step(2 * it + 1, 1)
        return c

    pairs = nch // 2
    lax.fori_loop(0, pairs, pair, 0)

    @pl.when(nch % 2 == 1)
    def _tail():
        step(2 * pairs, 0)

    for b in range(2):
        pltpu.make_async_copy(
            bufs[b]["ebuf"], p_hbm.at[pl.ds(0, CH)], bufs[b]["so"]).wait()


_p1_call = pl.kernel(
    _p1_body,
    out_type=jax.ShapeDtypeStruct((E,), jnp.float32),
    mesh=_mesh,
    compiler_params=_SC_PARAMS,
    scratch_types=[
        t for _ in range(2) for t in (
            pltpu.VMEM((CH,), jnp.int32),
            pltpu.VMEM((CH,), jnp.int32),
            pltpu.VMEM((CH, DH), jnp.bfloat16),
            pltpu.VMEM((CH, DH), jnp.bfloat16),
            pltpu.VMEM((CH,), jnp.float32),
        )
    ] + [pltpu.SemaphoreType.DMA] * 6,
)


def _p2_body(f2_hbm, src_hbm, dst_hbm, p_hbm, zer_hbm, zden_hbm,
             out_hbm, den_hbm, *s):
    bufs = []
    for b in range(2):
        o = b * 7
        bufs.append(dict(
            isrc=s[o], idst=s[o + 1], sdst=s[o + 2], pbuf=s[o + 3],
            sp=s[o + 4], rows=s[o + 5], wrows=s[o + 6],
            si=s[16 + b], sr=s[18 + b], ss=s[20 + b]))
    acc = s[14]
    den = s[15]

    cid = lax.axis_index("c")
    sid = lax.axis_index("s")
    coff = cid * N

    pltpu.sync_copy(zer_hbm.at[pl.ds(sid * RPS, RPS)],
                    acc.at[pl.ds(sid * RPS, RPS)])

    @pl.when(cid == 0)
    def _zero_den():
        pltpu.sync_copy(zden_hbm.at[pl.ds(sid * RPS, RPS)],
                        den.at[pl.ds(sid * RPS, RPS)])

    @pl.when(sid == NS - 1)
    def _zero_tail():
        pltpu.sync_copy(zer_hbm.at[pl.ds(NS * RPS, RTL)],
                        acc.at[pl.ds(NS * RPS, RTL)])

        @pl.when(cid == 0)
        def _zero_den_tail():
            pltpu.sync_copy(zden_hbm.at[pl.ds(NS * RPS, RTL)],
                            den.at[pl.ds(NS * RPS, RTL)])

    plsc.subcore_barrier()
    base = sid * EPW2

    def issue_idx(g, bb):
        off = base + g * CH2
        pltpu.async_copy(src_hbm.at[pl.ds(off, CH2)], bb["isrc"], bb["si"])
        pltpu.async_copy(dst_hbm.at[pl.ds(off, CH2)], bb["idst"], bb["si"])
        pltpu.async_copy(p_hbm.at[pl.ds(off, CH2)], bb["pbuf"], bb["si"])

    def launch_rows(bb):
        pltpu.make_async_copy(
            src_hbm.at[pl.ds(0, CH2)], bb["isrc"], bb["si"]).wait()
        pltpu.make_async_copy(
            dst_hbm.at[pl.ds(0, CH2)], bb["idst"], bb["si"]).wait()
        pltpu.make_async_copy(
            p_hbm.at[pl.ds(0, CH2)], bb["pbuf"], bb["si"]).wait()
        _vset(bb["isrc"], bb["isrc"], coff)
        pltpu.async_copy(f2_hbm.at[bb["isrc"]], bb["rows"], bb["sr"])

    def step(g, b):
        bb = bufs[b]
        nb = bufs[1 - b]
        pltpu.make_async_copy(
            f2_hbm.at[bb["isrc"]], bb["rows"], bb["sr"]).wait()

        @pl.when(g >= 2)
        def _drain_scatter():
            pltpu.make_async_copy(
                bb["wrows"], acc.at[bb["sdst"]], bb["ss"]).wait()

            @pl.when(cid == 0)
            def _drain_den():
                pltpu.make_async_copy(
                    bb["sp"], den.at[bb["sdst"]], bb["ss"]).wait()

        _vset(bb["sdst"], bb["idst"], 0)
        _vset(bb["sp"], bb["pbuf"], 0)

        @pl.when(g + 1 < NCH2)
        def _launch_next():
            launch_rows(nb)

        rows, wrows = bb["rows"], bb["wrows"]

        def wbody(jj, c2):
            pvec = bb["pbuf"][pl.ds(jj * LANES, LANES)]
            for l in range(LANES):
                i = jj * LANES + l
                pv = pvec[l]
                for k in range(DIM // LANES):
                    sl = pl.ds(k * LANES, LANES)
                    wrows[i, sl] = rows[i, sl] * pv
            return c2

        lax.fori_loop(0, CH2 // LANES, wbody, 0)

        @pl.when(g + 2 < NCH2)
        def _prefetch_idx():
            issue_idx(g + 2, bb)

        pltpu.async_copy(bb["wrows"], acc.at[bb["sdst"]], bb["ss"], add=True)

        @pl.when(cid == 0)
        def _scatter_den():
            pltpu.async_copy(bb["sp"], den.at[bb["sdst"]], bb["ss"],
                             add=True)

    issue_idx(jnp.int32(0), bufs[0])
    issue_idx(jnp.int32(1), bufs[1])
    launch_rows(bufs[0])

    def pair(it, c):
        step(2 * it, 0)
        step(2 * it + 1, 1)
        return c

    lax.fori_loop(0, NCH2 // 2, pair, 0)
    for b in range(2):
        pltpu.make_async_copy(
            bufs[b]["wrows"], acc.at[bufs[b]["sdst"]], bufs[b]["ss"]).wait()

        @pl.when(cid == 0)
        def _drain_den_tail():
            pltpu.make_async_copy(
                bufs[b]["sp"], den.at[bufs[b]["sdst"]], bufs[b]["ss"]).wait()

    plsc.subcore_barrier()
    pltpu.sync_copy(acc.at[pl.ds(sid * RPS, RPS)],
                    out_hbm.at[pl.ds(coff + sid * RPS, RPS)])

    @pl.when(cid == 0)
    def _den_out():
        pltpu.sync_copy(den.at[pl.ds(sid * RPS, RPS)],
                        den_hbm.at[pl.ds(sid * RPS, RPS)])

    @pl.when(sid == NS - 1)
    def _out_tail():
        pltpu.sync_copy(acc.at[pl.ds(NS * RPS, RTL)],
                        out_hbm.at[pl.ds(coff + NS * RPS, RTL)])

        @pl.when(cid == 0)
        def _den_out_tail():
            pltpu.sync_copy(den.at[pl.ds(NS * RPS, RTL)],
                            den_hbm.at[pl.ds(NS * RPS, RTL)])


_p2_call = pl.kernel(
    _p2_body,
    out_type=[
        jax.ShapeDtypeStruct((2 * N, DIM), jnp.float32),
        jax.ShapeDtypeStruct((N,), jnp.float32),
    ],
    mesh=_mesh,
    compiler_params=_SC_PARAMS,
    scratch_types=[
        t for _ in range(2) for t in (
            pltpu.VMEM((CH2,), jnp.int32),
            pltpu.VMEM((CH2,), jnp.int32),
            pltpu.VMEM((CH2,), jnp.int32),
            pltpu.VMEM((CH2,), jnp.float32),
            pltpu.VMEM((CH2,), jnp.float32),
            pltpu.VMEM((CH2, DIM), jnp.float32),
            pltpu.VMEM((CH2, DIM), jnp.float32),
        )
    ] + [pltpu.VMEM_SHARED((N, DIM), jnp.float32)]
      + [pltpu.VMEM_SHARED((N,), jnp.float32)]
      + [pltpu.SemaphoreType.DMA] * 6,
)


def _comb_body(a_ref, d_ref, o_ref):
    a0 = a_ref[0]
    a1 = a_ref[1]
    den = d_ref[...] + 1e-9
    o_ref[...] = jnp.maximum(a0, a1) / den


_comb_call = pl.pallas_call(
    _comb_body,
    grid=(1,),
    in_specs=[
        pl.BlockSpec((2, N, DIM), lambda i: (0, 0, 0)),
        pl.BlockSpec((N, 1), lambda i: (0, 0)),
    ],
    out_specs=pl.BlockSpec((N, DIM), lambda i: (0, 0)),
    out_shape=jax.ShapeDtypeStruct((N, DIM), jnp.float32),
)


def kernel(h_v, edge_index, fc_W, fc_b, pi_w):
    src = edge_index[0].astype(jnp.int32)
    dst = edge_index[1].astype(jnp.int32)
    f_parts, gbf, fbf = _mm_call(
        h_v, fc_W, fc_b.reshape(1, DH), pi_w.reshape(1, DH))
    f2 = f_parts.reshape(2 * N, DIM)
    p = _p1_call(gbf, fbf, src, dst)
    accf, den = _p2_call(f2, src, dst, p,
                         jnp.zeros((N, DIM), jnp.float32),
                         jnp.zeros((N,), jnp.float32))
    return _comb_call(accf.reshape(2, N, DIM), den.reshape(N, 1))
